# baseline (speedup 1.0000x reference)
import functools

import jax
import jax.numpy as jnp
from jax import lax
from jax.experimental import pallas as pl
from jax.experimental.pallas import tpu as pltpu
from jax.experimental.pallas import tpu_sc as plsc

_B = 4
_S = 4096
_D = 1024
_NC = 2
_NS = 16
_NW = _NC * _NS
_ROWS_PER_W = _S // _NW
_CH = 32
_NCHUNK = _ROWS_PER_W // _CH

_mesh = plsc.VectorSubcoreMesh(core_axis_name="c", subcore_axis_name="s")


@functools.partial(
    pl.kernel,
    mesh=_mesh,
    out_type=jax.ShapeDtypeStruct((_B, _S, _D), jnp.float32),
    scratch_types=[
        pltpu.SemaphoreType.DMA,
    ],
)
def _pe_hbm2hbm(table_hbm, out_hbm, sem):
    wid = lax.axis_index("s") * _NC + lax.axis_index("c")
    base = wid * _ROWS_PER_W
    handles = [
        pltpu.async_copy(
            table_hbm.at[pl.ds(base, _ROWS_PER_W)],
            out_hbm.at[b, pl.ds(base, _ROWS_PER_W)],
            sem)
        for b in range(_B)
    ]
    for h in handles:
        h.wait()


@functools.partial(
    pl.kernel,
    mesh=_mesh,
    out_type=jax.ShapeDtypeStruct((_B, _S, _D), jnp.float32),
    scratch_types=[
        pltpu.VMEM((2, _CH, _D), jnp.float32),
        pltpu.SemaphoreType.DMA((2,)),
        pltpu.SemaphoreType.DMA((2,)),
    ],
)
def _pe_broadcast(table_hbm, out_hbm, buf, load_sem, store_sem):
    wid = lax.axis_index("s") * _NC + lax.axis_index("c")
    base = wid * _ROWS_PER_W

    def load(i, slot):
        return pltpu.async_copy(
            table_hbm.at[pl.ds(base + i * _CH, _CH)],
            buf.at[slot],
            load_sem.at[slot])

    def store(i, slot, b):
        return pltpu.async_copy(
            buf.at[slot],
            out_hbm.at[b, pl.ds(base + i * _CH, _CH)],
            store_sem.at[slot])

    pending_stores = {0: [], 1: []}
    h = load(0, 0)
    for i in range(_NCHUNK):
        s = i % 2
        if i + 1 < _NCHUNK:
            ns = 1 - s
            for sh in pending_stores[ns]:
                sh.wait()
            pending_stores[ns] = []
            next_h = load(i + 1, ns)
        h.wait()
        pending_stores[s] = [store(i, s, b) for b in range(_B)]
        if i + 1 < _NCHUNK:
            h = next_h
    for s in (0, 1):
        for sh in pending_stores[s]:
            sh.wait()


def kernel(inputs, position_embeddings):
    del inputs
    return _pe_hbm2hbm(position_embeddings)


# speedup vs baseline: 43.2306x; 43.2306x over previous
import functools

import jax
import jax.numpy as jnp
from jax import lax
from jax.experimental import pallas as pl
from jax.experimental.pallas import tpu as pltpu
from jax.experimental.pallas import tpu_sc as plsc

_B = 4
_S = 4096
_D = 1024
_NC = 2
_NS = 16
_NW = _NC * _NS
_ROWS_PER_W = _S // _NW
_CH = 32
_NCHUNK = _ROWS_PER_W // _CH

_mesh = plsc.VectorSubcoreMesh(core_axis_name="c", subcore_axis_name="s")


_CH3 = 16
_NSLOT = 6
_NCHUNK3 = _ROWS_PER_W // _CH3


@functools.partial(
    pl.kernel,
    mesh=_mesh,
    out_type=jax.ShapeDtypeStruct((_B, _S, _D), jnp.float32),
    scratch_types=[
        pltpu.VMEM((_NSLOT, _CH3, _D), jnp.float32),
        pltpu.SemaphoreType.DMA((_NSLOT,)),
        pltpu.SemaphoreType.DMA((_NSLOT,)),
    ],
)
def _pe_ring(table_hbm, out_hbm, buf, load_sem, store_sem):
    wid = lax.axis_index("s") * _NC + lax.axis_index("c")
    base = wid * _ROWS_PER_W

    def load(i):
        return pltpu.async_copy(
            table_hbm.at[pl.ds(base + i * _CH3, _CH3)],
            buf.at[i % _NSLOT],
            load_sem.at[i % _NSLOT])

    def store(i, b):
        return pltpu.async_copy(
            buf.at[i % _NSLOT],
            out_hbm.at[b, pl.ds(base + i * _CH3, _CH3)],
            store_sem.at[i % _NSLOT])

    load_h = [None] * _NCHUNK3
    store_h = [None] * _NCHUNK3
    for j in range(min(_NSLOT, _NCHUNK3)):
        load_h[j] = load(j)
    for i in range(_NCHUNK3):
        if i > 0 and i - 1 + _NSLOT < _NCHUNK3:
            for sh in store_h[i - 1]:
                sh.wait()
            load_h[i - 1 + _NSLOT] = load(i - 1 + _NSLOT)
        load_h[i].wait()
        store_h[i] = [store(i, b) for b in range(_B)]
    waited = set(range(0, max(0, _NCHUNK3 - _NSLOT)))
    for i in range(_NCHUNK3):
        if i not in waited:
            for sh in store_h[i]:
                sh.wait()


@functools.partial(
    pl.kernel,
    mesh=_mesh,
    out_type=jax.ShapeDtypeStruct((_B, _S, _D), jnp.float32),
    scratch_types=[
        pltpu.VMEM((2, _CH, _D), jnp.float32),
        pltpu.SemaphoreType.DMA((2,)),
        pltpu.SemaphoreType.DMA((2,)),
    ],
)
def _pe_broadcast(table_hbm, out_hbm, buf, load_sem, store_sem):
    wid = lax.axis_index("s") * _NC + lax.axis_index("c")
    base = wid * _ROWS_PER_W

    def load(i, slot):
        return pltpu.async_copy(
            table_hbm.at[pl.ds(base + i * _CH, _CH)],
            buf.at[slot],
            load_sem.at[slot])

    def store(i, slot, b):
        return pltpu.async_copy(
            buf.at[slot],
            out_hbm.at[b, pl.ds(base + i * _CH, _CH)],
            store_sem.at[slot])

    pending_stores = {0: [], 1: []}
    h = load(0, 0)
    for i in range(_NCHUNK):
        s = i % 2
        if i + 1 < _NCHUNK:
            ns = 1 - s
            for sh in pending_stores[ns]:
                sh.wait()
            pending_stores[ns] = []
            next_h = load(i + 1, ns)
        h.wait()
        pending_stores[s] = [store(i, s, b) for b in range(_B)]
        if i + 1 < _NCHUNK:
            h = next_h
    for s in (0, 1):
        for sh in pending_stores[s]:
            sh.wait()


def kernel(inputs, position_embeddings):
    del inputs
    return _pe_ring(position_embeddings)


# speedup vs baseline: 44.2457x; 1.0235x over previous
import functools

import jax
import jax.numpy as jnp
from jax import lax
from jax.experimental import pallas as pl
from jax.experimental.pallas import tpu as pltpu
from jax.experimental.pallas import tpu_sc as plsc

_B = 4
_S = 4096
_D = 1024
_NC = 2
_NS = 16
_NW = _NC * _NS
_ROWS_PER_W = _S // _NW
_CH = 32
_NCHUNK = _ROWS_PER_W // _CH

_mesh = plsc.VectorSubcoreMesh(core_axis_name="c", subcore_axis_name="s")


_CH3 = 32
_NSLOT = 3
_NCHUNK3 = _ROWS_PER_W // _CH3


@functools.partial(
    pl.kernel,
    mesh=_mesh,
    out_type=jax.ShapeDtypeStruct((_B, _S, _D), jnp.float32),
    scratch_types=[
        pltpu.VMEM((_NSLOT, _CH3, _D), jnp.float32),
        pltpu.SemaphoreType.DMA((_NSLOT,)),
        pltpu.SemaphoreType.DMA((_NSLOT,)),
    ],
)
def _pe_ring(table_hbm, out_hbm, buf, load_sem, store_sem):
    wid = lax.axis_index("s") * _NC + lax.axis_index("c")
    base = wid * _ROWS_PER_W

    def load(i):
        return pltpu.async_copy(
            table_hbm.at[pl.ds(base + i * _CH3, _CH3)],
            buf.at[i % _NSLOT],
            load_sem.at[i % _NSLOT])

    def store(i, b):
        return pltpu.async_copy(
            buf.at[i % _NSLOT],
            out_hbm.at[b, pl.ds(base + i * _CH3, _CH3)],
            store_sem.at[i % _NSLOT])

    load_h = [None] * _NCHUNK3
    store_h = [None] * _NCHUNK3
    for j in range(min(_NSLOT, _NCHUNK3)):
        load_h[j] = load(j)
    for i in range(_NCHUNK3):
        if i > 0 and i - 1 + _NSLOT < _NCHUNK3:
            for sh in store_h[i - 1]:
                sh.wait()
            load_h[i - 1 + _NSLOT] = load(i - 1 + _NSLOT)
        load_h[i].wait()
        store_h[i] = [store(i, b) for b in range(_B)]
    waited = set(range(0, max(0, _NCHUNK3 - _NSLOT)))
    for i in range(_NCHUNK3):
        if i not in waited:
            for sh in store_h[i]:
                sh.wait()


@functools.partial(
    pl.kernel,
    mesh=_mesh,
    out_type=jax.ShapeDtypeStruct((_B, _S, _D), jnp.float32),
    scratch_types=[
        pltpu.VMEM((2, _CH, _D), jnp.float32),
        pltpu.SemaphoreType.DMA((2,)),
        pltpu.SemaphoreType.DMA((2,)),
    ],
)
def _pe_broadcast(table_hbm, out_hbm, buf, load_sem, store_sem):
    wid = lax.axis_index("s") * _NC + lax.axis_index("c")
    base = wid * _ROWS_PER_W

    def load(i, slot):
        return pltpu.async_copy(
            table_hbm.at[pl.ds(base + i * _CH, _CH)],
            buf.at[slot],
            load_sem.at[slot])

    def store(i, slot, b):
        return pltpu.async_copy(
            buf.at[slot],
            out_hbm.at[b, pl.ds(base + i * _CH, _CH)],
            store_sem.at[slot])

    pending_stores = {0: [], 1: []}
    h = load(0, 0)
    for i in range(_NCHUNK):
        s = i % 2
        if i + 1 < _NCHUNK:
            ns = 1 - s
            for sh in pending_stores[ns]:
                sh.wait()
            pending_stores[ns] = []
            next_h = load(i + 1, ns)
        h.wait()
        pending_stores[s] = [store(i, s, b) for b in range(_B)]
        if i + 1 < _NCHUNK:
            h = next_h
    for s in (0, 1):
        for sh in pending_stores[s]:
            sh.wait()


def kernel(inputs, position_embeddings):
    del inputs
    return _pe_ring(position_embeddings)
